# baseline (speedup 1.0000x reference)
import jax
import jax.numpy as jnp
from jax.experimental import pallas as pl
from jax.experimental.pallas import tpu as pltpu


_Q = 1024
_K = 64
_CB = 32
_H = 256


def _index_sort_kernel(q_ref, t_ref, dist_ref, idx_ref):
    d = jax.lax.dot_general(
        q_ref[...], t_ref[0],
        (((1,), (1,)), ((), ())),
        preferred_element_type=jnp.float32,
    )
    v = jnp.concatenate([d[_H * g : _H * (g + 1), :] for g in range(4)], axis=1)

    lane = jax.lax.broadcasted_iota(jnp.int32, v.shape, 1)
    r = jax.lax.broadcasted_iota(jnp.int32, v.shape, 0)
    row = r + _H * (lane // _CB)
    idx = row

    k = 2
    while k <= _Q:
        j = k // 2
        while j >= 1:
            if 8 <= j < _H:
                m = _H // (2 * j)
                v4 = v.reshape(m, 2, j, 128)
                i4 = idx.reshape(m, 2, j, 128)
                lo_v, hi_v = v4[:, 0], v4[:, 1]
                lo_i, hi_i = i4[:, 0], i4[:, 1]
                dir_up = ((row & k) == 0).reshape(m, 2, j, 128)[:, 0]
                lo_first = (lo_v < hi_v) | ((lo_v == hi_v) & (lo_i < hi_i))
                keep = dir_up == lo_first
                nlo_v = jnp.where(keep, lo_v, hi_v)
                nhi_v = jnp.where(keep, hi_v, lo_v)
                nlo_i = jnp.where(keep, lo_i, hi_i)
                nhi_i = jnp.where(keep, hi_i, lo_i)
                v = jnp.stack([nlo_v, nhi_v], axis=1).reshape(_H, 128)
                idx = jnp.stack([nlo_i, nhi_i], axis=1).reshape(_H, 128)
            else:
                if j < 8:
                    lower = (r & j) == 0
                    pv = jnp.where(lower, jnp.roll(v, -j, axis=0), jnp.roll(v, j, axis=0))
                    pi = jnp.where(lower, jnp.roll(idx, -j, axis=0), jnp.roll(idx, j, axis=0))
                elif j == _H:
                    glow = (lane & _CB) == 0
                    pv = jnp.where(glow, jnp.roll(v, -_CB, axis=1), jnp.roll(v, _CB, axis=1))
                    pi = jnp.where(glow, jnp.roll(idx, -_CB, axis=1), jnp.roll(idx, _CB, axis=1))
                else:
                    pv = jnp.roll(v, 64, axis=1)
                    pi = jnp.roll(idx, 64, axis=1)
                is_lower = (row & j) == 0
                dir_up = (row & k) == 0
                a_first = (v < pv) | ((v == pv) & (idx < pi))
                keep_a = (is_lower == dir_up) == a_first
                v = jnp.where(keep_a, v, pv)
                idx = jnp.where(keep_a, idx, pi)
            j //= 2
        k *= 2

    for g in range(4):
        dist_ref[0, _H * g : _H * (g + 1), :] = v[:, _CB * g : _CB * (g + 1)]
        idx_ref[0, _H * g : _H * (g + 1), :] = idx[:, _CB * g : _CB * (g + 1)]


def kernel(query, index, k):
    tail = jax.lax.dynamic_slice_in_dim(index, index.shape[0] - k, _K, axis=0)
    tail3 = tail.reshape(_K // _CB, _CB, 32)
    dist, idx = pl.pallas_call(
        _index_sort_kernel,
        grid=(_K // _CB,),
        in_specs=[
            pl.BlockSpec((_Q, 32), lambda i: (0, 0)),
            pl.BlockSpec((1, _CB, 32), lambda i: (i, 0, 0)),
        ],
        out_specs=(
            pl.BlockSpec((1, _Q, _CB), lambda i: (i, 0, 0)),
            pl.BlockSpec((1, _Q, _CB), lambda i: (i, 0, 0)),
        ),
        out_shape=(
            jax.ShapeDtypeStruct((_K // _CB, _Q, _CB), jnp.float32),
            jax.ShapeDtypeStruct((_K // _CB, _Q, _CB), jnp.int32),
        ),
        compiler_params=pltpu.CompilerParams(
            dimension_semantics=("parallel",),
        ),
    )(query, tail3)
    return (
        jnp.concatenate([dist[0], dist[1]], axis=1),
        jnp.concatenate([idx[0], idx[1]], axis=1),
    )


# speedup vs baseline: 1.4178x; 1.4178x over previous
import jax
import jax.numpy as jnp
from jax.experimental import pallas as pl


_Q = 1024
_K = 64


def _index_sort_kernel(q_ref, t_ref, dist_ref, idx_ref):
    d = jax.lax.dot_general(
        q_ref[...], t_ref[...],
        (((1,), (1,)), ((), ())),
        preferred_element_type=jnp.float32,
    )
    h = _Q // 2
    v = jnp.concatenate([d[:h, :], d[h:, :]], axis=1)

    lane = jax.lax.broadcasted_iota(jnp.int32, v.shape, 1)
    r = jax.lax.broadcasted_iota(jnp.int32, v.shape, 0)
    row = r + jnp.where(lane >= _K, h, 0)
    idx = row

    k = 2
    while k <= _Q:
        j = k // 2
        while j >= 1:
            if 8 <= j < h:
                m = h // (2 * j)
                v4 = v.reshape(m, 2, j, 128)
                i4 = idx.reshape(m, 2, j, 128)
                lo_v, hi_v = v4[:, 0], v4[:, 1]
                lo_i, hi_i = i4[:, 0], i4[:, 1]
                lo_first = (lo_v < hi_v) | ((lo_v == hi_v) & (lo_i < hi_i))
                if k == _Q:
                    keep = lo_first
                elif k == h:
                    lane_p = jax.lax.broadcasted_iota(jnp.int32, lo_v.shape, 2)
                    keep = (lane_p < _K) == lo_first
                else:
                    b = jax.lax.broadcasted_iota(jnp.int32, lo_v.shape, 0)
                    keep = ((b & (k // (2 * j))) == 0) == lo_first
                nlo_v = jnp.where(keep, lo_v, hi_v)
                nhi_v = jnp.where(keep, hi_v, lo_v)
                nlo_i = jnp.where(keep, lo_i, hi_i)
                nhi_i = jnp.where(keep, hi_i, lo_i)
                v = jnp.stack([nlo_v, nhi_v], axis=1).reshape(h, 128)
                idx = jnp.stack([nlo_i, nhi_i], axis=1).reshape(h, 128)
            elif j < 8:
                v3 = v.reshape(h // 8, 8, 128)
                i3 = idx.reshape(h // 8, 8, 128)
                s = jax.lax.broadcasted_iota(jnp.int32, v3.shape, 1)
                lower3 = (s & j) == 0
                pv = jnp.where(lower3, jnp.roll(v3, -j, axis=1), jnp.roll(v3, j, axis=1))
                pi = jnp.where(lower3, jnp.roll(i3, -j, axis=1), jnp.roll(i3, j, axis=1))
                a_first = (v3 < pv) | ((v3 == pv) & (i3 < pi))
                if k == _Q:
                    keep_a = lower3 == a_first
                elif k == h:
                    lane3 = jax.lax.broadcasted_iota(jnp.int32, v3.shape, 2)
                    keep_a = (lower3 == (lane3 < _K)) == a_first
                elif k >= 8:
                    b3 = jax.lax.broadcasted_iota(jnp.int32, v3.shape, 0)
                    keep_a = (lower3 == ((b3 & (k // 8)) == 0)) == a_first
                else:
                    keep_a = (lower3 == ((s & k) == 0)) == a_first
                v = jnp.where(keep_a, v3, pv).reshape(h, 128)
                idx = jnp.where(keep_a, i3, pi).reshape(h, 128)
            else:
                pv = jnp.roll(v, _K, axis=1)
                pi = jnp.roll(idx, _K, axis=1)
                is_lower = lane < _K
                a_first = (v < pv) | ((v == pv) & (idx < pi))
                keep_a = is_lower == a_first
                v = jnp.where(keep_a, v, pv)
                idx = jnp.where(keep_a, idx, pi)
            j //= 2
        k *= 2

    dist_ref[: h, :] = v[:, :_K]
    dist_ref[h:, :] = v[:, _K:]
    idx_ref[: h, :] = idx[:, :_K]
    idx_ref[h:, :] = idx[:, _K:]


def kernel(query, index, k):
    tail = jax.lax.dynamic_slice_in_dim(index, index.shape[0] - k, _K, axis=0)
    return pl.pallas_call(
        _index_sort_kernel,
        out_shape=(
            jax.ShapeDtypeStruct((query.shape[0], _K), jnp.float32),
            jax.ShapeDtypeStruct((query.shape[0], _K), jnp.int32),
        ),
    )(query, tail)


# speedup vs baseline: 1.4377x; 1.0140x over previous
import jax
import jax.numpy as jnp
from jax.experimental import pallas as pl


_Q = 1024
_K = 64


def _index_sort_kernel(q_ref, t_ref, dist_ref, idx_ref):
    d = jax.lax.dot_general(
        q_ref[...], t_ref[...],
        (((1,), (1,)), ((), ())),
        preferred_element_type=jnp.float32,
    )
    h = _Q // 2
    v = jnp.concatenate([d[:h, :], d[h:, :]], axis=1)

    lane = jax.lax.broadcasted_iota(jnp.int32, v.shape, 1)
    r = jax.lax.broadcasted_iota(jnp.int32, v.shape, 0)
    row = r + jnp.where(lane >= _K, h, 0)
    idx = row

    k = 2
    while k <= _Q:
        j = k // 2
        while j >= 1:
            if 8 <= j < h:
                m = h // (2 * j)
                v4 = v.reshape(m, 2, j, 128)
                i4 = idx.reshape(m, 2, j, 128)
                lo_v, hi_v = v4[:, 0], v4[:, 1]
                lo_i, hi_i = i4[:, 0], i4[:, 1]
                lo_first = (lo_v < hi_v) | ((lo_v == hi_v) & (lo_i < hi_i))
                if k == _Q:
                    keep = lo_first
                elif k == h:
                    lane_p = jax.lax.broadcasted_iota(jnp.int32, lo_v.shape, 2)
                    keep = (lane_p < _K) == lo_first
                else:
                    b = jax.lax.broadcasted_iota(jnp.int32, lo_v.shape, 0)
                    keep = ((b & (k // (2 * j))) == 0) == lo_first
                nlo_v = jnp.where(keep, lo_v, hi_v)
                nhi_v = jnp.where(keep, hi_v, lo_v)
                nlo_i = jnp.where(keep, lo_i, hi_i)
                nhi_i = jnp.where(keep, hi_i, lo_i)
                v = jnp.stack([nlo_v, nhi_v], axis=1).reshape(h, 128)
                idx = jnp.stack([nlo_i, nhi_i], axis=1).reshape(h, 128)
            elif j < 8:
                v3 = v.reshape(h // 8, 8, 128)
                i3 = idx.reshape(h // 8, 8, 128)
                s = jax.lax.broadcasted_iota(jnp.int32, v3.shape, 1)
                lower3 = (s & j) == 0
                perm = s ^ j
                pv = jnp.take_along_axis(v3, perm, axis=1)
                pi = jnp.take_along_axis(i3, perm, axis=1)
                a_first = (v3 < pv) | ((v3 == pv) & (i3 < pi))
                if k == _Q:
                    keep_a = lower3 == a_first
                elif k == h:
                    lane3 = jax.lax.broadcasted_iota(jnp.int32, v3.shape, 2)
                    keep_a = (lower3 == (lane3 < _K)) == a_first
                elif k >= 8:
                    b3 = jax.lax.broadcasted_iota(jnp.int32, v3.shape, 0)
                    keep_a = (lower3 == ((b3 & (k // 8)) == 0)) == a_first
                else:
                    keep_a = (lower3 == ((s & k) == 0)) == a_first
                v = jnp.where(keep_a, v3, pv).reshape(h, 128)
                idx = jnp.where(keep_a, i3, pi).reshape(h, 128)
            else:
                pv = jnp.roll(v, _K, axis=1)
                pi = jnp.roll(idx, _K, axis=1)
                is_lower = lane < _K
                a_first = (v < pv) | ((v == pv) & (idx < pi))
                keep_a = is_lower == a_first
                v = jnp.where(keep_a, v, pv)
                idx = jnp.where(keep_a, idx, pi)
            j //= 2
        k *= 2

    dist_ref[: h, :] = v[:, :_K]
    dist_ref[h:, :] = v[:, _K:]
    idx_ref[: h, :] = idx[:, :_K]
    idx_ref[h:, :] = idx[:, _K:]


def kernel(query, index, k):
    tail = jax.lax.dynamic_slice_in_dim(index, index.shape[0] - k, _K, axis=0)
    return pl.pallas_call(
        _index_sort_kernel,
        out_shape=(
            jax.ShapeDtypeStruct((query.shape[0], _K), jnp.float32),
            jax.ShapeDtypeStruct((query.shape[0], _K), jnp.int32),
        ),
    )(query, tail)
